# Initial kernel scaffold; baseline (speedup 1.0000x reference)
import jax
import jax.numpy as jnp
from jax.experimental import pallas as pl


def kernel(x, ResampleMap):
    raise NotImplementedError("write your pallas kernel here")


# speedup vs baseline: 336.1449x; 336.1449x over previous
import functools

import jax
import jax.numpy as jnp
from jax import lax
from jax.experimental import pallas as pl
from jax.experimental.pallas import tpu as pltpu
from jax.experimental.pallas import tpu_sc as plsc

IN_DIM = 100000
OUT_DIM = 100000
NN = 32

NC = 2
NS = 16
LANES = 16
NW = NC * NS

ROWS_PER_W = OUT_DIM // NW
IDX_PER_W = ROWS_PER_W * NN
GROUPS = -(-ROWS_PER_W // LANES)
OUT_PAD = GROUPS * LANES
GPC = 14
CHUNKS = -(-GROUPS // GPC)
CHUNK_LEN = GPC * LANES * NN
LAST_BASE = IDX_PER_W - CHUNK_LEN

_mesh = plsc.VectorSubcoreMesh(core_axis_name="c", subcore_axis_name="s")


@functools.partial(
    pl.kernel,
    mesh=_mesh,
    out_type=jax.ShapeDtypeStruct((NW, OUT_PAD), jnp.float32),
    scratch_types=[
        pltpu.VMEM((IN_DIM,), jnp.float32),
        pltpu.VMEM((CHUNK_LEN,), jnp.int32),
        pltpu.VMEM((OUT_PAD,), jnp.float32),
    ],
    compiler_params=pltpu.CompilerParams(needs_layout_passes=False),
)
def _resample_sc(x_hbm, map_hbm, out_hbm, table_v, chunk_v, out_v):
    wid = lax.axis_index("s") * NC + lax.axis_index("c")
    pltpu.sync_copy(x_hbm, table_v)
    map_base = wid * IDX_PER_W
    lane = lax.broadcasted_iota(jnp.int32, (LANES,), 0)

    def chunk_body(cidx, carry):
        base_rel = jnp.minimum(cidx * CHUNK_LEN, LAST_BASE)
        pltpu.sync_copy(
            map_hbm.at[pl.ds(map_base + base_rel, CHUNK_LEN)], chunk_v)

        def group_body(k, carry2):
            g = cidx * GPC + k
            bvec = jnp.minimum(
                g * (LANES * NN) - base_rel + NN * lane, CHUNK_LEN - NN)
            acc = jnp.zeros((LANES,), jnp.float32)
            for j in range(NN):
                inds = plsc.load_gather(chunk_v, [bvec + j])
                vals = plsc.load_gather(table_v, [inds])
                acc = acc + vals
            out_v[pl.ds(g * LANES, LANES)] = acc * (1.0 / NN)
            return carry2

        lax.fori_loop(0, GPC, group_body, 0)
        return carry

    lax.fori_loop(0, CHUNKS, chunk_body, 0)
    pltpu.sync_copy(out_v, out_hbm.at[wid])


def kernel(x, ResampleMap):
    out = _resample_sc(x.reshape(IN_DIM), ResampleMap)
    return out[:, :ROWS_PER_W].reshape(1, OUT_DIM)


# speedup vs baseline: 352.6172x; 1.0490x over previous
import functools

import jax
import jax.numpy as jnp
from jax import lax
from jax.experimental import pallas as pl
from jax.experimental.pallas import tpu as pltpu
from jax.experimental.pallas import tpu_sc as plsc

IN_DIM = 100000
OUT_DIM = 100000
NN = 32

NC = 2
NS = 16
LANES = 16
NW = NC * NS

ROWS_PER_W = OUT_DIM // NW
IDX_PER_W = ROWS_PER_W * NN
GROUPS = -(-ROWS_PER_W // LANES)
OUT_PAD = GROUPS * LANES
GPC = 14
CHUNKS = -(-GROUPS // GPC)
CHUNK_LEN = GPC * LANES * NN
LAST_BASE = IDX_PER_W - CHUNK_LEN

_mesh = plsc.VectorSubcoreMesh(core_axis_name="c", subcore_axis_name="s")


@functools.partial(
    pl.kernel,
    mesh=_mesh,
    out_type=jax.ShapeDtypeStruct((NW, OUT_PAD), jnp.float32),
    scratch_types=[
        pltpu.VMEM((IN_DIM,), jnp.float32),
        pltpu.VMEM((CHUNK_LEN,), jnp.int32),
        pltpu.VMEM((CHUNK_LEN,), jnp.int32),
        pltpu.VMEM((OUT_PAD,), jnp.float32),
        pltpu.SemaphoreType.DMA,
        pltpu.SemaphoreType.DMA,
        pltpu.SemaphoreType.DMA,
    ],
    compiler_params=pltpu.CompilerParams(needs_layout_passes=False),
)
def _resample_sc(x_hbm, map_hbm, out_hbm,
                 table_v, buf0, buf1, out_v, sem_t, sem0, sem1):
    wid = lax.axis_index("s") * NC + lax.axis_index("c")
    map_base = wid * IDX_PER_W
    lane = lax.broadcasted_iota(jnp.int32, (LANES,), 0)
    bufs = (buf0, buf1)
    sems = (sem0, sem1)

    table_cp = pltpu.async_copy(x_hbm, table_v, sem_t)
    pending = pltpu.async_copy(
        map_hbm.at[pl.ds(map_base, CHUNK_LEN)], buf0, sem0)
    table_cp.wait()

    for c in range(CHUNKS):
        base_rel = min(c * CHUNK_LEN, LAST_BASE)
        buf = bufs[c % 2]
        nxt = None
        if c + 1 < CHUNKS:
            nbase = min((c + 1) * CHUNK_LEN, LAST_BASE)
            nxt = pltpu.async_copy(
                map_hbm.at[pl.ds(map_base + nbase, CHUNK_LEN)],
                bufs[(c + 1) % 2], sems[(c + 1) % 2])
        pending.wait()

        def group_body(k, carry, buf=buf, off=c * CHUNK_LEN - base_rel):
            bvec = jnp.minimum(
                k * (LANES * NN) + off + NN * lane, CHUNK_LEN - NN)
            acc = jnp.zeros((LANES,), jnp.float32)
            for j in range(NN):
                inds = plsc.load_gather(buf, [bvec + j])
                vals = plsc.load_gather(table_v, [inds])
                acc = acc + vals
            out_v[pl.ds((c * GPC + k) * LANES, LANES)] = acc * (1.0 / NN)
            return carry

        lax.fori_loop(0, GPC, group_body, 0)
        pending = nxt

    pltpu.sync_copy(out_v, out_hbm.at[wid])


def kernel(x, ResampleMap):
    out = _resample_sc(x.reshape(IN_DIM), ResampleMap)
    return out[:, :ROWS_PER_W].reshape(1, OUT_DIM)


# speedup vs baseline: 574.8299x; 1.6302x over previous
import functools

import jax
import jax.numpy as jnp
from jax import lax
from jax.experimental import pallas as pl
from jax.experimental.pallas import tpu as pltpu
from jax.experimental.pallas import tpu_sc as plsc

IN_DIM = 100000
OUT_DIM = 100000
NN = 32

NC = 2
NS = 16
LANES = 16
NW = NC * NS

ROWS_PER_W = OUT_DIM // NW
IDX_PER_W = ROWS_PER_W * NN
GROUPS = -(-ROWS_PER_W // LANES)
OUT_PAD = GROUPS * LANES
GPC = 14
CHUNKS = -(-GROUPS // GPC)
CHUNK_LEN = GPC * LANES * NN
LAST_BASE = IDX_PER_W - CHUNK_LEN

_mesh = plsc.VectorSubcoreMesh(core_axis_name="c", subcore_axis_name="s")


@functools.partial(
    pl.kernel,
    mesh=_mesh,
    out_type=jax.ShapeDtypeStruct((NW, OUT_PAD), jnp.float32),
    scratch_types=[
        pltpu.VMEM((IN_DIM,), jnp.float32),
        pltpu.VMEM((CHUNK_LEN,), jnp.int32),
        pltpu.VMEM((CHUNK_LEN,), jnp.int32),
        pltpu.VMEM((OUT_PAD,), jnp.float32),
        pltpu.SemaphoreType.DMA,
        pltpu.SemaphoreType.DMA,
        pltpu.SemaphoreType.DMA,
    ],
    compiler_params=pltpu.CompilerParams(needs_layout_passes=False),
)
def _resample_sc(x_hbm, map_hbm, out_hbm,
                 table_v, buf0, buf1, out_v, sem_t, sem0, sem1):
    wid = lax.axis_index("s") * NC + lax.axis_index("c")
    map_base = wid * IDX_PER_W
    lane = lax.broadcasted_iota(jnp.int32, (LANES,), 0)
    bufs = (buf0, buf1)
    sems = (sem0, sem1)

    table_cp = pltpu.async_copy(x_hbm, table_v, sem_t)
    pending = pltpu.async_copy(
        map_hbm.at[pl.ds(map_base, CHUNK_LEN)], buf0, sem0)
    table_cp.wait()

    for c in range(CHUNKS):
        base_rel = min(c * CHUNK_LEN, LAST_BASE)
        buf = bufs[c % 2]
        nxt = None
        if c + 1 < CHUNKS:
            nbase = min((c + 1) * CHUNK_LEN, LAST_BASE)
            nxt = pltpu.async_copy(
                map_hbm.at[pl.ds(map_base + nbase, CHUNK_LEN)],
                bufs[(c + 1) % 2], sems[(c + 1) % 2])
        pending.wait()

        def group_body(k, carry, buf=buf, off=c * CHUNK_LEN - base_rel):
            bvec = jnp.minimum(
                k * (LANES * NN) + off + NN * lane, CHUNK_LEN - NN)
            acc = jnp.zeros((LANES,), jnp.float32)
            for j in range(NN):
                rot = (lane + j) & (NN - 1)
                inds = plsc.load_gather(buf, [bvec + rot])
                vals = plsc.load_gather(table_v, [inds])
                acc = acc + vals
            out_v[pl.ds((c * GPC + k) * LANES, LANES)] = acc * (1.0 / NN)
            return carry

        lax.fori_loop(0, GPC, group_body, 0)
        pending = nxt

    pltpu.sync_copy(out_v, out_hbm.at[wid])


def kernel(x, ResampleMap):
    out = _resample_sc(x.reshape(IN_DIM), ResampleMap)
    return out[:, :ROWS_PER_W].reshape(1, OUT_DIM)


# speedup vs baseline: 575.5987x; 1.0013x over previous
import functools

import jax
import jax.numpy as jnp
from jax import lax
from jax.experimental import pallas as pl
from jax.experimental.pallas import tpu as pltpu
from jax.experimental.pallas import tpu_sc as plsc

IN_DIM = 100000
OUT_DIM = 100000
NN = 32

NC = 2
NS = 16
LANES = 16
NW = NC * NS

GPC = 14
CHUNKS = 14
GROUPS_MAIN = GPC * CHUNKS
ROWS_MAIN = GROUPS_MAIN * LANES
ROWS_LAST = OUT_DIM - (NW - 1) * ROWS_MAIN
GROUPS_LAST = ROWS_LAST // LANES
CHUNK_LEN = GPC * LANES * NN
IDX_MAIN = ROWS_MAIN * NN
IDX_LAST = ROWS_LAST * NN

_mesh = plsc.VectorSubcoreMesh(core_axis_name="c", subcore_axis_name="s")


@functools.partial(
    pl.kernel,
    mesh=_mesh,
    out_type=jax.ShapeDtypeStruct((OUT_DIM,), jnp.float32),
    scratch_types=[
        pltpu.VMEM((IN_DIM,), jnp.float32),
        pltpu.VMEM((CHUNK_LEN,), jnp.int32),
        pltpu.VMEM((CHUNK_LEN,), jnp.int32),
        pltpu.VMEM((ROWS_MAIN,), jnp.float32),
        pltpu.SemaphoreType.DMA,
        pltpu.SemaphoreType.DMA,
        pltpu.SemaphoreType.DMA,
    ],
    compiler_params=pltpu.CompilerParams(needs_layout_passes=False),
)
def _resample_sc(x_hbm, map_hbm, out_hbm,
                 table_v, buf0, buf1, out_v, sem_t, sem0, sem1):
    wid = lax.axis_index("s") * NC + lax.axis_index("c")
    is_last = wid == NW - 1
    groups_w = jnp.where(is_last, GROUPS_LAST, GROUPS_MAIN)
    idx_end = jnp.where(is_last, IDX_LAST - CHUNK_LEN, IDX_MAIN - CHUNK_LEN)
    map_base = wid * IDX_MAIN
    lane = lax.broadcasted_iota(jnp.int32, (LANES,), 0)
    bufs = (buf0, buf1)
    sems = (sem0, sem1)

    table_cp = pltpu.async_copy(x_hbm, table_v, sem_t)
    pending = pltpu.async_copy(
        map_hbm.at[pl.ds(map_base, CHUNK_LEN)], buf0, sem0)
    table_cp.wait()

    for c in range(CHUNKS):
        base_rel = jnp.minimum(c * CHUNK_LEN, idx_end)
        buf = bufs[c % 2]
        nxt = None
        if c + 1 < CHUNKS:
            nbase = jnp.minimum((c + 1) * CHUNK_LEN, idx_end)
            nxt = pltpu.async_copy(
                map_hbm.at[pl.ds(map_base + nbase, CHUNK_LEN)],
                bufs[(c + 1) % 2], sems[(c + 1) % 2])
        pending.wait()
        off = c * CHUNK_LEN - base_rel
        n_groups = jnp.clip(groups_w - c * GPC, 0, GPC)

        def group_body(k, carry, buf=buf, off=off, c=c):
            bvec = jnp.minimum(
                k * (LANES * NN) + off + NN * lane, CHUNK_LEN - NN)
            acc = jnp.zeros((LANES,), jnp.float32)
            for j in range(NN):
                rot = (lane + j) & (NN - 1)
                inds = plsc.load_gather(buf, [bvec + rot])
                vals = plsc.load_gather(table_v, [inds])
                acc = acc + vals
            out_v[pl.ds((c * GPC + k) * LANES, LANES)] = acc * (1.0 / NN)
            return carry

        lax.fori_loop(0, n_groups, group_body, 0)
        pending = nxt

    out_base = wid * ROWS_MAIN
    pltpu.sync_copy(out_v.at[pl.ds(0, ROWS_LAST)],
                    out_hbm.at[pl.ds(out_base, ROWS_LAST)])

    @pl.when(jnp.logical_not(is_last))
    def _copy_tail():
        pltpu.sync_copy(
            out_v.at[pl.ds(ROWS_LAST, ROWS_MAIN - ROWS_LAST)],
            out_hbm.at[pl.ds(out_base + ROWS_LAST, ROWS_MAIN - ROWS_LAST)])


def kernel(x, ResampleMap):
    return _resample_sc(x.reshape(IN_DIM), ResampleMap).reshape(1, OUT_DIM)


# speedup vs baseline: 637.8528x; 1.1082x over previous
import functools

import jax
import jax.numpy as jnp
from jax import lax
from jax.experimental import pallas as pl
from jax.experimental.pallas import tpu as pltpu
from jax.experimental.pallas import tpu_sc as plsc

IN_DIM = 100000
OUT_DIM = 100000
NN = 32

NC = 2
NS = 16
LANES = 16
NW = NC * NS

GPC = 20
CHUNKS = 10
GROUPS_MAIN = GPC * CHUNKS
ROWS_MAIN = GROUPS_MAIN * LANES
ROWS_LAST = OUT_DIM - (NW - 1) * ROWS_MAIN
GROUPS_LAST = ROWS_LAST // LANES
CHUNK_LEN = GPC * LANES * NN
IDX_MAIN = ROWS_MAIN * NN
IDX_LAST = ROWS_LAST * NN

_mesh = plsc.VectorSubcoreMesh(core_axis_name="c", subcore_axis_name="s")


@functools.partial(
    pl.kernel,
    mesh=_mesh,
    out_type=jax.ShapeDtypeStruct((1, OUT_DIM), jnp.float32),
    scratch_types=[
        pltpu.VMEM((IN_DIM,), jnp.float32),
        pltpu.VMEM((CHUNK_LEN,), jnp.int32),
        pltpu.VMEM((CHUNK_LEN,), jnp.int32),
        pltpu.VMEM((ROWS_MAIN,), jnp.float32),
        pltpu.SemaphoreType.DMA,
        pltpu.SemaphoreType.DMA,
        pltpu.SemaphoreType.DMA,
    ],
    compiler_params=pltpu.CompilerParams(needs_layout_passes=False),
)
def _resample_sc(x_hbm, map_hbm, out_hbm,
                 table_v, buf0, buf1, out_v, sem_t, sem0, sem1):
    wid = lax.axis_index("s") * NC + lax.axis_index("c")
    is_last = wid == NW - 1
    groups_w = jnp.where(is_last, GROUPS_LAST, GROUPS_MAIN)
    idx_end = jnp.where(is_last, IDX_LAST - CHUNK_LEN, IDX_MAIN - CHUNK_LEN)
    map_base = wid * IDX_MAIN
    lane = lax.broadcasted_iota(jnp.int32, (LANES,), 0)
    bufs = (buf0, buf1)
    sems = (sem0, sem1)

    table_cp = pltpu.async_copy(x_hbm.at[0], table_v, sem_t)
    pending = pltpu.async_copy(
        map_hbm.at[pl.ds(map_base, CHUNK_LEN)], buf0, sem0)
    table_cp.wait()

    for c in range(CHUNKS):
        base_rel = jnp.minimum(c * CHUNK_LEN, idx_end)
        buf = bufs[c % 2]
        nxt = None
        if c + 1 < CHUNKS:
            nbase = jnp.minimum((c + 1) * CHUNK_LEN, idx_end)
            nxt = pltpu.async_copy(
                map_hbm.at[pl.ds(map_base + nbase, CHUNK_LEN)],
                bufs[(c + 1) % 2], sems[(c + 1) % 2])
        pending.wait()
        off = c * CHUNK_LEN - base_rel
        n_groups = jnp.clip(groups_w - c * GPC, 0, GPC)

        def group_body(k, carry, buf=buf, off=off, c=c):
            bvec = jnp.minimum(
                k * (LANES * NN) + off + NN * lane, CHUNK_LEN - NN)
            acc = jnp.zeros((LANES,), jnp.float32)
            for j in range(NN):
                rot = (lane + j) & (NN - 1)
                inds = plsc.load_gather(buf, [bvec + rot])
                vals = plsc.load_gather(table_v, [inds])
                acc = acc + vals
            out_v[pl.ds((c * GPC + k) * LANES, LANES)] = acc * (1.0 / NN)
            return carry

        lax.fori_loop(0, n_groups, group_body, 0)
        pending = nxt

    @pl.when(jnp.logical_not(is_last))
    def _copy_main():
        pltpu.sync_copy(out_v, out_hbm.at[0, pl.ds(wid * ROWS_MAIN, ROWS_MAIN)])

    @pl.when(is_last)
    def _copy_last():
        pltpu.sync_copy(out_v.at[pl.ds(0, ROWS_LAST)],
                        out_hbm.at[0, pl.ds((NW - 1) * ROWS_MAIN, ROWS_LAST)])


def kernel(x, ResampleMap):
    return _resample_sc(x, ResampleMap)


# speedup vs baseline: 708.6918x; 1.1111x over previous
import functools

import jax
import jax.numpy as jnp
from jax import lax
from jax.experimental import pallas as pl
from jax.experimental.pallas import tpu as pltpu
from jax.experimental.pallas import tpu_sc as plsc

IN_DIM = 100000
OUT_DIM = 100000
NN = 32

NC = 2
NS = 16
LANES = 16
NW = NC * NS

GPC = 20
CHUNKS = 10
GROUPS_MAIN = GPC * CHUNKS
ROWS_MAIN = GROUPS_MAIN * LANES
ROWS_LAST = OUT_DIM - (NW - 1) * ROWS_MAIN
GROUPS_LAST = ROWS_LAST // LANES
CHUNK_LEN = GPC * LANES * NN
IDX_MAIN = ROWS_MAIN * NN
IDX_LAST = ROWS_LAST * NN

_mesh = plsc.VectorSubcoreMesh(core_axis_name="c", subcore_axis_name="s")


@functools.partial(
    pl.kernel,
    mesh=_mesh,
    out_type=jax.ShapeDtypeStruct((1, OUT_DIM), jnp.float32),
    scratch_types=[
        pltpu.VMEM((IN_DIM,), jnp.float32),
        pltpu.VMEM((CHUNK_LEN,), jnp.int32),
        pltpu.VMEM((CHUNK_LEN,), jnp.int32),
        pltpu.VMEM((ROWS_MAIN,), jnp.float32),
        pltpu.SemaphoreType.DMA,
        pltpu.SemaphoreType.DMA,
        pltpu.SemaphoreType.DMA,
    ],
    compiler_params=pltpu.CompilerParams(needs_layout_passes=False),
)
def _resample_sc(x_hbm, map_hbm, out_hbm,
                 table_v, buf0, buf1, out_v, sem_t, sem0, sem1):
    wid = lax.axis_index("s") * NC + lax.axis_index("c")
    is_last = wid == NW - 1
    groups_w = jnp.where(is_last, GROUPS_LAST, GROUPS_MAIN)
    idx_end = jnp.where(is_last, IDX_LAST - CHUNK_LEN, IDX_MAIN - CHUNK_LEN)
    map_base = wid * IDX_MAIN
    lane = lax.broadcasted_iota(jnp.int32, (LANES,), 0)
    bufs = (buf0, buf1)
    sems = (sem0, sem1)

    table_cp = pltpu.async_copy(x_hbm.at[0], table_v, sem_t)
    pending = pltpu.async_copy(
        map_hbm.at[pl.ds(map_base, CHUNK_LEN)], buf0, sem0)
    table_cp.wait()
    del pending

    def run_chunk(c, buf):
        base_rel = jnp.minimum(c * CHUNK_LEN, idx_end)
        off = c * CHUNK_LEN - base_rel
        n_groups = jnp.clip(groups_w - c * GPC, 0, GPC)

        def group_body(k, carry):
            bvec = jnp.minimum(
                k * (LANES * NN) + off + NN * lane, CHUNK_LEN - NN)
            acc = jnp.zeros((LANES,), jnp.float32)
            for j in range(NN):
                rot = (lane + j) & (NN - 1)
                inds = plsc.load_gather(buf, [bvec + rot])
                vals = plsc.load_gather(table_v, [inds])
                acc = acc + vals
            out_v[pl.ds((c * GPC + k) * LANES, LANES)] = acc * (1.0 / NN)
            return carry

        lax.fori_loop(0, n_groups, group_body, 0)

    def prefetch(c, buf, sem):
        base = jnp.minimum(c * CHUNK_LEN, idx_end)
        return pltpu.async_copy(
            map_hbm.at[pl.ds(map_base + base, CHUNK_LEN)], buf, sem)

    def chunk_pair(p, carry):
        c0 = 2 * p
        prefetch(c0 + 1, buf1, sem1)
        pltpu.make_async_copy(
            map_hbm.at[pl.ds(map_base, CHUNK_LEN)], buf0, sem0).wait()
        run_chunk(c0, buf0)
        prefetch(c0 + 2, buf0, sem0)
        pltpu.make_async_copy(
            map_hbm.at[pl.ds(map_base, CHUNK_LEN)], buf1, sem1).wait()
        run_chunk(c0 + 1, buf1)
        return carry

    lax.fori_loop(0, CHUNKS // 2, chunk_pair, 0)
    pltpu.make_async_copy(
        map_hbm.at[pl.ds(map_base, CHUNK_LEN)], buf0, sem0).wait()

    @pl.when(jnp.logical_not(is_last))
    def _copy_main():
        pltpu.sync_copy(out_v, out_hbm.at[0, pl.ds(wid * ROWS_MAIN, ROWS_MAIN)])

    @pl.when(is_last)
    def _copy_last():
        pltpu.sync_copy(out_v.at[pl.ds(0, ROWS_LAST)],
                        out_hbm.at[0, pl.ds((NW - 1) * ROWS_MAIN, ROWS_LAST)])


def kernel(x, ResampleMap):
    return _resample_sc(x, ResampleMap)
